# baseline (speedup 1.0000x reference)
import functools
import jax
import jax.numpy as jnp
from jax import lax
from jax.experimental import pallas as pl
from jax.experimental.pallas import tpu as pltpu
from jax.experimental.pallas import tpu_sc as plsc

NC = 2
NS = 16
L = 16
NW = NC * NS
R = 16


def _sc_add(x_hbm, pos_hbm, out_hbm, xbuf, pbuf):
    seqd = pos_hbm.shape[0]
    nbatch = x_hbm.shape[0] // seqd
    elems_per_w = seqd // NW
    chunk = R * 1024
    wid = lax.axis_index("s") * NC + lax.axis_index("c")
    pos0 = wid * elems_per_w
    nvec = chunk // L

    def chunk_body(c, carry):
        posb = pos0 + c * chunk
        pltpu.sync_copy(pos_hbm.at[pl.ds(posb, chunk)], pbuf)

        def batch_body(b, carry2):
            base = b * seqd + posb
            pltpu.sync_copy(x_hbm.at[pl.ds(base, chunk)], xbuf)

            def vec_body(j, carry3):
                sl = pl.ds(j * L, L)
                xbuf[sl] = xbuf[sl] + pbuf[sl]
                return carry3

            lax.fori_loop(0, nvec, vec_body, 0)
            pltpu.sync_copy(xbuf, out_hbm.at[pl.ds(base, chunk)])
            return carry2

        lax.fori_loop(0, nbatch, batch_body, 0)
        return carry

    lax.fori_loop(0, elems_per_w // chunk, chunk_body, 0)


def kernel(x, pos_table):
    b, s, d = x.shape
    xf = x.reshape(b * s * d)
    posf = pos_table.reshape(s * d)
    mesh = plsc.VectorSubcoreMesh(core_axis_name="c", subcore_axis_name="s")
    run = functools.partial(
        pl.kernel,
        mesh=mesh,
        out_type=jax.ShapeDtypeStruct((b * s * d,), jnp.float32),
        scratch_types=[
            pltpu.VMEM((R * 1024,), jnp.float32),
            pltpu.VMEM((R * 1024,), jnp.float32),
        ],
    )(_sc_add)
    return run(xf, posf).reshape(b, s, d)


# speedup vs baseline: 1.8607x; 1.8607x over previous
import functools
import jax
import jax.numpy as jnp
from jax import lax
from jax.experimental import pallas as pl
from jax.experimental.pallas import tpu as pltpu
from jax.experimental.pallas import tpu_sc as plsc

NC = 2
NS = 16
L = 16
NW = NC * NS
R = 16

S_BLK = 2048
TC_B = 3


def _tc_add(x_ref, pos_ref, o_ref):
    o_ref[...] = x_ref[...] + pos_ref[...]


def _tc_call(x, pos_table):
    batch, seq_len, d_model = x.shape
    n_s = seq_len // S_BLK
    return pl.pallas_call(
        _tc_add,
        grid=(n_s, batch),
        in_specs=[
            pl.BlockSpec((1, S_BLK, d_model), lambda s, b: (b, s, 0)),
            pl.BlockSpec((S_BLK, d_model), lambda s, b: (s, 0)),
        ],
        out_specs=pl.BlockSpec((1, S_BLK, d_model), lambda s, b: (b, s, 0)),
        out_shape=jax.ShapeDtypeStruct((batch, seq_len, d_model), x.dtype),
    )(x, pos_table)


def _sc_add(x_hbm, pos_hbm, out_hbm, xbuf, pbuf):
    seqd = pos_hbm.shape[0]
    nbatch = x_hbm.shape[0] // seqd
    elems_per_w = seqd // NW
    chunk = R * 1024
    wid = lax.axis_index("s") * NC + lax.axis_index("c")
    pos0 = wid * elems_per_w
    nvec = chunk // L

    def chunk_body(c, carry):
        posb = pos0 + c * chunk
        pltpu.sync_copy(pos_hbm.at[pl.ds(posb, chunk)], pbuf)

        def batch_body(b, carry2):
            base = b * seqd + posb
            pltpu.sync_copy(x_hbm.at[pl.ds(base, chunk)], xbuf)

            def vec_body(j, carry3):
                sl = pl.ds(j * L, L)
                xbuf[sl] = xbuf[sl] + pbuf[sl]
                return carry3

            lax.fori_loop(0, nvec, vec_body, 0)
            pltpu.sync_copy(xbuf, out_hbm.at[pl.ds(base, chunk)])
            return carry2

        lax.fori_loop(0, nbatch, batch_body, 0)
        return carry

    lax.fori_loop(0, elems_per_w // chunk, chunk_body, 0)


def _sc_call(x, pos_table):
    b, s, d = x.shape
    xf = x.reshape(b * s * d)
    posf = pos_table.reshape(s * d)
    mesh = plsc.VectorSubcoreMesh(core_axis_name="c", subcore_axis_name="s")
    run = functools.partial(
        pl.kernel,
        mesh=mesh,
        out_type=jax.ShapeDtypeStruct((b * s * d,), jnp.float32),
        scratch_types=[
            pltpu.VMEM((R * 1024,), jnp.float32),
            pltpu.VMEM((R * 1024,), jnp.float32),
        ],
    )(_sc_add)
    return run(xf, posf).reshape(b, s, d)


def kernel(x, pos_table):
    out_tc = _tc_call(x[:TC_B], pos_table)
    out_sc = _sc_call(x[TC_B:], pos_table)
    return jnp.concatenate([out_tc, out_sc], axis=0)


# speedup vs baseline: 1.8626x; 1.0010x over previous
import functools
import jax
import jax.numpy as jnp
from jax import lax
from jax.experimental import pallas as pl
from jax.experimental.pallas import tpu as pltpu
from jax.experimental.pallas import tpu_sc as plsc

NC = 2
NS = 16
L = 16
NW = NC * NS
R = 16

S_BLK = 2048
TC_B = 3


def _tc_add(x_ref, pos_ref, o_ref):
    o_ref[...] = x_ref[...] + pos_ref[...]


def _tc_call(x, pos_table):
    batch, seq_len, d_model = x.shape
    n_s = seq_len // S_BLK
    return pl.pallas_call(
        _tc_add,
        grid=(n_s, batch),
        in_specs=[
            pl.BlockSpec((1, S_BLK, d_model), lambda s, b: (b, s, 0)),
            pl.BlockSpec((S_BLK, d_model), lambda s, b: (s, 0)),
        ],
        out_specs=pl.BlockSpec((1, S_BLK, d_model), lambda s, b: (b, s, 0)),
        out_shape=jax.ShapeDtypeStruct((batch, seq_len, d_model), x.dtype),
    )(x, pos_table)


def _sc_add(x_hbm, pos_hbm, out_hbm, xbuf, pbuf):
    seqd = pos_hbm.shape[0]
    nbatch = x_hbm.shape[0] // seqd
    elems_per_w = seqd // NW
    chunk = R * 1024
    wid = lax.axis_index("s") * NC + lax.axis_index("c")
    pos0 = wid * elems_per_w
    nvec = chunk // L

    def chunk_body(c, carry):
        posb = pos0 + c * chunk
        pltpu.sync_copy(pos_hbm.at[pl.ds(posb, chunk)], pbuf)

        def batch_body(b, carry2):
            base = b * seqd + posb
            pltpu.sync_copy(x_hbm.at[pl.ds(base, chunk)], xbuf)

            def vec_body(j, carry3):
                sl = pl.ds(j * L, L)
                xbuf[sl] = xbuf[sl] + pbuf[sl]
                return carry3

            lax.fori_loop(0, nvec, vec_body, 0)
            pltpu.sync_copy(xbuf, out_hbm.at[pl.ds(base, chunk)])
            return carry2

        lax.fori_loop(0, nbatch, batch_body, 0)
        return carry

    lax.fori_loop(0, elems_per_w // chunk, chunk_body, 0)


def _sc_call(x, pos_table):
    b, s, d = x.shape
    xf = x.reshape(b * s * d)
    posf = pos_table.reshape(s * d)
    mesh = plsc.VectorSubcoreMesh(core_axis_name="c", subcore_axis_name="s")
    run = functools.partial(
        pl.kernel,
        mesh=mesh,
        compiler_params=pltpu.CompilerParams(use_tc_tiling_on_sc=True),
        out_type=jax.ShapeDtypeStruct((b * s * d,), jnp.float32),
        scratch_types=[
            pltpu.VMEM((R * 1024,), jnp.float32),
            pltpu.VMEM((R * 1024,), jnp.float32),
        ],
    )(_sc_add)
    return run(xf, posf).reshape(b, s, d)


def kernel(x, pos_table):
    out_tc = _tc_call(x[:TC_B], pos_table)
    out_sc = _sc_call(x[TC_B:], pos_table)
    return jnp.concatenate([out_tc, out_sc], axis=0)


# speedup vs baseline: 3.8587x; 2.0717x over previous
import functools
import jax
import jax.numpy as jnp
from jax import lax
from jax.experimental import pallas as pl
from jax.experimental.pallas import tpu as pltpu
from jax.experimental.pallas import tpu_sc as plsc

NC = 2
NS = 16
L = 16
NW = NC * NS
R = 8


def _sc_add(x_hbm, pos_hbm, out_hbm,
            pA, pB, a0, a1, a2, a3, b0, b1, b2, b3,
            spA, spB, slA, slB, ssA, ssB):
    seq = pos_hbm.shape[0]
    d = pos_hbm.shape[1]
    nb = x_hbm.shape[0] // seq
    pos_rows = seq // NW
    nchunks = pos_rows // R
    nvec = d // L
    wid = lax.axis_index("s") * NC + lax.axis_index("c")
    pos_base = wid * pos_rows
    QA = (a0, a1, a2, a3)
    QB = (b0, b1, b2, b3)

    def issue(chunk, quad, pbuf, sl, sp):
        pr = pos_base + chunk * R
        pltpu.async_copy(pos_hbm.at[pl.ds(pr, R)], pbuf, sp)
        for b in range(nb):
            pltpu.async_copy(x_hbm.at[pl.ds(b * seq + pr, R)], quad[b], sl)

    def wait_loads(quad, pbuf, sl, sp):
        pltpu.make_async_copy(pos_hbm.at[pl.ds(0, R)], pbuf, sp).wait()
        for b in range(nb):
            pltpu.make_async_copy(x_hbm.at[pl.ds(0, R)], quad[b], sl).wait()

    def drain_stores(quad, ss):
        for b in range(nb):
            pltpu.make_async_copy(x_hbm.at[pl.ds(0, R)], quad[b], ss).wait()

    def compute(quad, pbuf):
        def row(r, carry):
            for k in range(nvec):
                sl_ = pl.ds(k * L, L)
                pv = pbuf[r, sl_]
                for b in range(nb):
                    quad[b][r, sl_] = quad[b][r, sl_] + pv
            return carry
        lax.fori_loop(0, R, row, 0)

    def store(chunk, quad, ss):
        pr = pos_base + chunk * R
        for b in range(nb):
            pltpu.async_copy(quad[b], out_hbm.at[pl.ds(b * seq + pr, R)], ss)

    issue(0, QA, pA, slA, spA)

    def body(i, carry):
        cA = 2 * i
        cB = 2 * i + 1

        wait_loads(QA, pA, slA, spA)
        compute(QA, pA)
        store(cA, QA, ssA)

        @pl.when(i > 0)
        def _():
            drain_stores(QB, ssB)
        issue(cB, QB, pB, slB, spB)

        wait_loads(QB, pB, slB, spB)
        compute(QB, pB)
        store(cB, QB, ssB)

        drain_stores(QA, ssA)

        @pl.when(i < nchunks // 2 - 1)
        def _():
            issue(cA + 2, QA, pA, slA, spA)
        return carry

    lax.fori_loop(0, nchunks // 2, body, 0)
    drain_stores(QB, ssB)


def kernel(x, pos_table):
    b, s, d = x.shape
    xf = x.reshape(b * s, d)
    mesh = plsc.VectorSubcoreMesh(core_axis_name="c", subcore_axis_name="s")
    run = functools.partial(
        pl.kernel,
        mesh=mesh,
        out_type=jax.ShapeDtypeStruct((b * s, d), jnp.float32),
        scratch_types=(
            [pltpu.VMEM((R, d), jnp.float32)] * 10
            + [pltpu.SemaphoreType.DMA] * 6
        ),
    )(_sc_add)
    return run(xf, pos_table).reshape(b, s, d)
